# baseline (speedup 1.0000x reference)
import functools

import jax
import jax.numpy as jnp
from jax import lax
from jax.experimental import pallas as pl
from jax.experimental.pallas import tpu as pltpu
from jax.experimental.pallas import tpu_sc as plsc

_B, _N, _D = 4096, 50, 256
_HID = 2 * _D
_OUT = 256

_NC, _NS = 2, 16
_NW = _NC * _NS
_B_PER_W = _B // _NW


@functools.cache
def _make_sc_gather():
    mesh = plsc.VectorSubcoreMesh(core_axis_name="c", subcore_axis_name="s")

    @functools.partial(
        pl.kernel,
        mesh=mesh,
        out_type=jax.ShapeDtypeStruct((_B, _D), jnp.float32),
        scratch_types=[
            pltpu.VMEM((_B_PER_W,), jnp.int32),
            pltpu.VMEM((_B_PER_W, _D), jnp.float32),
            pltpu.SemaphoreType.DMA,
        ],
    )
    def gather_k(table_hbm, idx_hbm, out_hbm, idx_v, rows_v, sem):
        wid = lax.axis_index("s") * _NC + lax.axis_index("c")
        base = wid * _B_PER_W
        pltpu.sync_copy(idx_hbm.at[pl.ds(base, _B_PER_W)], idx_v)
        pltpu.async_copy(table_hbm.at[idx_v], rows_v, sem).wait()
        pltpu.sync_copy(rows_v, out_hbm.at[pl.ds(base, _B_PER_W)])

    return gather_k


_BB = 256


def _mean_body(xa_ref, xb_ref, w1b_ref, b1_ref, o_ref):
    s = jnp.sum(xa_ref[...], axis=0) + jnp.sum(xb_ref[...], axis=0)
    mean = s * (1.0 / _N)
    o_ref[...] = jnp.dot(mean, w1b_ref[...],
                         preferred_element_type=jnp.float32) + b1_ref[...]


def _mean_call(xt, W1b, b1):
    return pl.pallas_call(
        _mean_body,
        grid=(_B // _BB,),
        in_specs=[
            pl.BlockSpec((_N // 2, _BB, _D), lambda i: (0, i, 0)),
            pl.BlockSpec((_N // 2, _BB, _D), lambda i: (1, i, 0)),
            pl.BlockSpec((_D, _HID), lambda i: (0, 0)),
            pl.BlockSpec((1, _HID), lambda i: (0, 0)),
        ],
        out_specs=pl.BlockSpec((_BB, _HID), lambda i: (i, 0)),
        out_shape=jax.ShapeDtypeStruct((_B, _HID), jnp.float32),
    )(xt, xt, W1b, b1.reshape(1, _HID))


_BB2 = 512


def _mlp_body(local_ref, p_ref, w1a_ref, w2_ref, b2_ref, o_ref):
    h = jnp.dot(local_ref[...], w1a_ref[...],
                preferred_element_type=jnp.float32)
    h = jnp.maximum(h + p_ref[...], 0.0)
    h = jnp.dot(h, w2_ref[...], preferred_element_type=jnp.float32)
    o_ref[...] = jnp.maximum(h + b2_ref[...], 0.0)


def _mlp_call(local, p, W1a, W2, b2):
    return pl.pallas_call(
        _mlp_body,
        grid=(_B // _BB2,),
        in_specs=[
            pl.BlockSpec((_BB2, _D), lambda i: (i, 0)),
            pl.BlockSpec((_BB2, _HID), lambda i: (i, 0)),
            pl.BlockSpec((_D, _HID), lambda i: (0, 0)),
            pl.BlockSpec((_HID, _OUT), lambda i: (0, 0)),
            pl.BlockSpec((1, _OUT), lambda i: (0, 0)),
        ],
        out_specs=pl.BlockSpec((_BB2, _OUT), lambda i: (i, 0)),
        out_shape=jax.ShapeDtypeStruct((_B, _OUT), jnp.float32),
    )(local, p, W1a, W2, b2.reshape(1, _OUT))


def kernel(x, edge_index, agent_nodes, W1, b1, W2, b2):
    a = agent_nodes.astype(jnp.int32)
    xt = jnp.transpose(x, (1, 0, 2))
    table = xt.reshape(_N * _B, _D)
    idx = a[:, 1] * _B + a[:, 0]
    local = _make_sc_gather()(table, idx)
    p = _mean_call(xt, W1[_D:], b1)
    return _mlp_call(local, p, W1[:_D], W2, b2)


# speedup vs baseline: 1.0758x; 1.0758x over previous
import functools

import jax
import jax.numpy as jnp
from jax import lax
from jax.experimental import pallas as pl
from jax.experimental.pallas import tpu as pltpu
from jax.experimental.pallas import tpu_sc as plsc

_B, _N, _D = 4096, 50, 256
_HID = 2 * _D
_OUT = 256

_NC, _NS, _NL = 2, 16, 16
_NW = _NC * _NS
_B_PER_W = _B // _NW


@functools.cache
def _make_sc_gather():
    mesh = plsc.VectorSubcoreMesh(core_axis_name="c", subcore_axis_name="s")

    @functools.partial(
        pl.kernel,
        mesh=mesh,
        out_type=jax.ShapeDtypeStruct((_B, _D), jnp.float32),
        scratch_types=[
            pltpu.VMEM((_B_PER_W,), jnp.int32),
            pltpu.VMEM((_B_PER_W,), jnp.int32),
            pltpu.VMEM((_B_PER_W,), jnp.int32),
            pltpu.VMEM((_B_PER_W, _D), jnp.float32),
            pltpu.SemaphoreType.DMA,
        ],
    )
    def gather_k(table_hbm, at_hbm, out_hbm, a0_v, a1_v, idx_v, rows_v, sem):
        wid = lax.axis_index("s") * _NC + lax.axis_index("c")
        base = wid * _B_PER_W
        pltpu.sync_copy(at_hbm.at[0, pl.ds(base, _B_PER_W)], a0_v)
        pltpu.sync_copy(at_hbm.at[1, pl.ds(base, _B_PER_W)], a1_v)
        for k in range(_B_PER_W // _NL):
            sl = pl.ds(k * _NL, _NL)
            idx_v[sl] = a1_v[sl] * _B + a0_v[sl]
        pltpu.async_copy(table_hbm.at[idx_v], rows_v, sem).wait()
        pltpu.sync_copy(rows_v, out_hbm.at[pl.ds(base, _B_PER_W)])

    return gather_k


_BB = 256


def _tc_body(xa_ref, xb_ref, local_ref, w1_ref, b1_ref, w2_ref, b2_ref, o_ref):
    s = jnp.sum(xa_ref[...], axis=0) + jnp.sum(xb_ref[...], axis=0)
    mean = s * (1.0 / _N)
    h = jnp.concatenate([local_ref[...], mean], axis=1)
    h = jnp.dot(h, w1_ref[...], preferred_element_type=jnp.float32)
    h = jnp.maximum(h + b1_ref[...], 0.0)
    h = jnp.dot(h, w2_ref[...], preferred_element_type=jnp.float32)
    o_ref[...] = jnp.maximum(h + b2_ref[...], 0.0)


def _tc_call(xt, local, W1, b1, W2, b2):
    return pl.pallas_call(
        _tc_body,
        grid=(_B // _BB,),
        in_specs=[
            pl.BlockSpec((_N // 2, _BB, _D), lambda i: (0, i, 0)),
            pl.BlockSpec((_N // 2, _BB, _D), lambda i: (1, i, 0)),
            pl.BlockSpec((_BB, _D), lambda i: (i, 0)),
            pl.BlockSpec((_HID, _HID), lambda i: (0, 0)),
            pl.BlockSpec((1, _HID), lambda i: (0, 0)),
            pl.BlockSpec((_HID, _OUT), lambda i: (0, 0)),
            pl.BlockSpec((1, _OUT), lambda i: (0, 0)),
        ],
        out_specs=pl.BlockSpec((_BB, _OUT), lambda i: (i, 0)),
        out_shape=jax.ShapeDtypeStruct((_B, _OUT), jnp.float32),
    )(xt, xt, local, W1, b1.reshape(1, _HID), W2, b2.reshape(1, _OUT))


def kernel(x, edge_index, agent_nodes, W1, b1, W2, b2):
    at = jnp.transpose(agent_nodes.astype(jnp.int32))
    xt = jnp.transpose(x, (1, 0, 2))
    table = xt.reshape(_N * _B, _D)
    local = _make_sc_gather()(table, at)
    return _tc_call(xt, local, W1, b1, W2, b2)
